# baseline (speedup 1.0000x reference)
import jax
import jax.numpy as jnp
from jax import lax
from jax.experimental import pallas as pl
from jax.experimental.pallas import tpu as pltpu
from jax.experimental.pallas import tpu_sc as plsc

_WEIGHT = 0.7
_L = 16


def _mse_alive_body(data_hbm, out_hbm, d_s, o_s):
    @pl.when(lax.axis_index("c") == 0)
    def _():
        pltpu.sync_copy(data_hbm, d_s)
        x0, x1, t0, t1 = d_s[0], d_s[1], d_s[2], d_s[3]
        a0, a1, p0, p1 = d_s[4], d_s[5], d_s[6], d_s[7]

        sq0 = (x0 - t0) * (x0 - t0)
        sq1 = (x1 - t1) * (x1 - t1)
        cv0 = (p0 == 2.0) & ((x0 < t0) | (a0 == 0.0))
        cv1 = (p1 == 2.0) & ((x1 < t1) | (a1 == 0.0))
        cp0 = p0 == 1.0
        cp1 = p1 == 1.0

        one = jnp.float32(1.0)
        zero = jnp.float32(0.0)
        half = jnp.float32(0.5)
        valid_count = jnp.where(cv0, one, zero) + jnp.where(cv1, one, zero)
        valid_sum = jnp.where(cv0, sq0, zero) + jnp.where(cv1, sq1, zero)
        pseudo_count = jnp.where(cp0, one, zero) + jnp.where(cp1, one, zero)
        pseudo_sum = jnp.where(cp0, sq0, zero) + jnp.where(cp1, sq1, zero)

        loss_true = jnp.where(
            valid_count > zero,
            valid_sum * jnp.where(valid_count == 2.0, half, one),
            zero,
        )
        loss_pseudo = jnp.where(
            pseudo_count > zero,
            pseudo_sum * jnp.where(pseudo_count == 2.0, half, one),
            zero,
        )
        loss = loss_true * _WEIGHT + loss_pseudo * (1.0 - _WEIGHT)

        o_s[0] = loss
        pltpu.sync_copy(o_s, out_hbm)


def kernel(inputs, target, target_label, alive, pseudo, bins):
    x = jnp.reshape(inputs, (-1,))[:2]
    t = target
    a = alive.astype(jnp.float32)
    p = pseudo.astype(jnp.float32)
    data = jnp.pad(jnp.concatenate([x, t, a, p]), (0, _L - 8))

    run = pl.kernel(
        _mse_alive_body,
        mesh=plsc.ScalarSubcoreMesh(axis_name="c", num_cores=1),
        out_type=jax.ShapeDtypeStruct((1,), jnp.float32),
        scratch_types=[
            pltpu.SMEM((_L,), jnp.float32),
            pltpu.SMEM((1,), jnp.float32),
        ],
    )
    out = run(data)
    return out[0]


# speedup vs baseline: 1.0290x; 1.0290x over previous
import jax
import jax.numpy as jnp
from jax import lax
from jax.experimental import pallas as pl
from jax.experimental.pallas import tpu as pltpu
from jax.experimental.pallas import tpu_sc as plsc

_WEIGHT = 0.7
_L = 16


def _mse_alive_body(data_hbm, out_hbm, d_s, o_s):
    pltpu.sync_copy(data_hbm, d_s)
    x0, x1, t0, t1 = d_s[0], d_s[1], d_s[2], d_s[3]
    a0, a1, p0, p1 = d_s[4], d_s[5], d_s[6], d_s[7]

    sq0 = (x0 - t0) * (x0 - t0)
    sq1 = (x1 - t1) * (x1 - t1)
    cv0 = (p0 == 2.0) & ((x0 < t0) | (a0 == 0.0))
    cv1 = (p1 == 2.0) & ((x1 < t1) | (a1 == 0.0))
    cp0 = p0 == 1.0
    cp1 = p1 == 1.0

    one = jnp.float32(1.0)
    zero = jnp.float32(0.0)
    half = jnp.float32(0.5)
    valid_count = jnp.where(cv0, one, zero) + jnp.where(cv1, one, zero)
    valid_sum = jnp.where(cv0, sq0, zero) + jnp.where(cv1, sq1, zero)
    pseudo_count = jnp.where(cp0, one, zero) + jnp.where(cp1, one, zero)
    pseudo_sum = jnp.where(cp0, sq0, zero) + jnp.where(cp1, sq1, zero)

    loss_true = jnp.where(
        valid_count > zero,
        valid_sum * jnp.where(valid_count == 2.0, half, one),
        zero,
    )
    loss_pseudo = jnp.where(
        pseudo_count > zero,
        pseudo_sum * jnp.where(pseudo_count == 2.0, half, one),
        zero,
    )
    loss = loss_true * _WEIGHT + loss_pseudo * (1.0 - _WEIGHT)

    o_s[0] = loss
    pltpu.sync_copy(o_s, out_hbm)


def kernel(inputs, target, target_label, alive, pseudo, bins):
    x = jnp.reshape(inputs, (-1,))[:2]
    t = target
    a = alive.astype(jnp.float32)
    p = pseudo.astype(jnp.float32)
    data = jnp.pad(jnp.concatenate([x, t, a, p]), (0, _L - 8))

    run = pl.kernel(
        _mse_alive_body,
        mesh=plsc.ScalarSubcoreMesh(axis_name="c", num_cores=1),
        out_type=jax.ShapeDtypeStruct((1,), jnp.float32),
        scratch_types=[
            pltpu.SMEM((_L,), jnp.float32),
            pltpu.SMEM((1,), jnp.float32),
        ],
    )
    out = run(data)
    return out[0]
